# Initial kernel scaffold; baseline (speedup 1.0000x reference)
import jax
import jax.numpy as jnp
from jax.experimental import pallas as pl


def kernel(input, W, b, ts_table, dow_table, adaptive):
    raise NotImplementedError("write your pallas kernel here")


# speedup vs baseline: 2.2416x; 2.2416x over previous
import functools

import jax
import jax.numpy as jnp
from jax import lax
from jax.experimental import pallas as pl
from jax.experimental.pallas import tpu as pltpu
from jax.experimental.pallas import tpu_sc as plsc

_B, _L, _N, _C = 8, 12, 2048, 1
_DIM = 24
_ADIM = 80
_TS = 24 * 12
_DOW = 7
_TOT = _B * _L * _N
_OUT_D = 3 * _DIM + _ADIM
_LN = _L * _N
_NC = 2
_NS = 16
_NW = _NC * _NS
_TPW = _TOT // _NW
_T = 512
_NCH = _TPW // _T
_NG = _T // 16
_NSUB = _T // 128


def _sc_body(inp_ref, ctab_ref, wb_ref, adp_ref, out_ref,
             inp_v, cidx_v, femb_v, rows_v, adp_v, wb_v, sem, gsem):
    wid = lax.axis_index("s") * _NC + lax.axis_index("c")
    pltpu.sync_copy(wb_ref, wb_v)

    lane = lax.iota(jnp.int32, 16)
    lane3 = lane * 3
    wds = [plsc.load_gather(wb_v, [jnp.full((16,), 1 + d, jnp.int32)])
           for d in range(_DIM)]
    bds = [plsc.load_gather(wb_v, [jnp.full((16,), 1 + _DIM + d, jnp.int32)])
           for d in range(_DIM)]

    def chunk_body(c, carry):
        t0 = wid * _TPW + c * _T
        pltpu.sync_copy(inp_ref.at[pl.ds(t0 * 3, _T * 3)], inp_v)

        arow0 = lax.rem(t0, _LN)
        adp_cp = pltpu.async_copy(adp_ref.at[pl.ds(arow0, _T)], adp_v, sem)

        for g in range(_NG):
            base = g * 48
            feat = plsc.load_gather(inp_v, [lane3 + base])
            tsv = plsc.load_gather(inp_v, [lane3 + (base + 1)])
            dwv = plsc.load_gather(inp_v, [lane3 + (base + 2)])
            comb = tsv.astype(jnp.int32) * _DOW + dwv.astype(jnp.int32)
            cidx_v[g // 8, pl.ds((g % 8) * 16, 16)] = comb
            tok16 = jnp.full((16,), g * 16, jnp.int32) + lane
            for d in range(_DIM):
                val = feat * wds[d] + bds[d]
                plsc.store_scatter(
                    femb_v, [tok16, jnp.full((16,), d, jnp.int32)], val)

        gcps = []
        for k in range(_NSUB):
            gcps.append(pltpu.async_copy(
                ctab_ref.at[cidx_v.at[k]],
                rows_v.at[pl.ds(k * 128, 128)], gsem))

        pltpu.sync_copy(femb_v, out_ref.at[pl.ds(t0, _T), pl.ds(0, _DIM)])

        adp_cp.wait()
        pltpu.sync_copy(adp_v, out_ref.at[pl.ds(t0, _T), pl.ds(72, _ADIM)])

        for cp in gcps:
            cp.wait()
        pltpu.sync_copy(rows_v, out_ref.at[pl.ds(t0, _T), pl.ds(24, 48)])
        return carry

    lax.fori_loop(0, _NCH, chunk_body, 0)


@jax.jit
def kernel(input, W, b, ts_table, dow_table, adaptive):
    inp_flat = input.reshape(-1)
    wb = jnp.concatenate([jnp.zeros((1,), jnp.float32),
                          W.reshape(-1), b])
    ctab = jnp.concatenate([
        jnp.broadcast_to(ts_table[:, None, :], (_TS, _DOW, _DIM)),
        jnp.broadcast_to(dow_table[None, :, :], (_TS, _DOW, _DIM)),
    ], axis=-1).reshape(_TS * _DOW, 2 * _DIM)
    adp_flat = adaptive.reshape(_LN, _ADIM)

    mesh = plsc.VectorSubcoreMesh(core_axis_name="c", subcore_axis_name="s")
    fn = pl.kernel(
        _sc_body,
        out_type=jax.ShapeDtypeStruct((_TOT, _OUT_D), jnp.float32),
        mesh=mesh,
        compiler_params=pltpu.CompilerParams(use_tc_tiling_on_sc=False,
                                              needs_layout_passes=False),
        scratch_types=[
            pltpu.VMEM((_T * 3,), jnp.float32),
            pltpu.VMEM((_NSUB, 128), jnp.int32),
            pltpu.VMEM((_T, _DIM), jnp.float32),
            pltpu.VMEM((_T, 48), jnp.float32),
            pltpu.VMEM((_T, _ADIM), jnp.float32),
            pltpu.VMEM((2 * _DIM + 1,), jnp.float32),
            pltpu.SemaphoreType.DMA,
            pltpu.SemaphoreType.DMA,
        ],
    )
    out = fn(inp_flat, ctab, wb, adp_flat)
    return out.reshape(_B, _L, _N, _OUT_D)
